# baseline (speedup 1.0000x reference)
import functools

import jax
import jax.numpy as jnp
from jax import lax
from jax.experimental import pallas as pl
from jax.experimental.pallas import tpu as pltpu

N = 10000
E = 320000
D = 128
NP = 10240
BLK = 1024
E2 = E + N
NW = 32
CB = 128
EPW = ((E2 + NW * CB - 1) // (NW * CB)) * CB
EP = EPW * NW


def _pre_body(x_ref, w_ref, att_ref, h_ref, a_ref):
    h = jnp.dot(x_ref[...], w_ref[...], preferred_element_type=jnp.float32)
    h_ref[...] = h
    a_ref[...] = jnp.dot(h, att_ref[...], preferred_element_type=jnp.float32)


def _tc_pre(x_pad, w, att_m):
    grid = (NP // BLK,)
    return pl.pallas_call(
        _pre_body,
        grid=grid,
        in_specs=[
            pl.BlockSpec((BLK, D), lambda i: (i, 0)),
            pl.BlockSpec((D, D), lambda i: (0, 0)),
            pl.BlockSpec((D, D), lambda i: (0, 0)),
        ],
        out_specs=[
            pl.BlockSpec((BLK, D), lambda i: (i, 0)),
            pl.BlockSpec((BLK, D), lambda i: (i, 0)),
        ],
        out_shape=[
            jax.ShapeDtypeStruct((NP, D), jnp.float32),
            jax.ShapeDtypeStruct((NP, D), jnp.float32),
        ],
    )(x_pad, w, att_m)


def _mid_body(p_ref, b_ref, w_ref, att_ref, h_ref, a_ref):
    s = p_ref[0] + p_ref[1] + b_ref[...]
    hin = jnp.maximum(s, 0.0)
    h = jnp.dot(hin, w_ref[...], preferred_element_type=jnp.float32)
    h_ref[...] = h
    a_ref[...] = jnp.dot(h, att_ref[...], preferred_element_type=jnp.float32)


def _tc_mid(acc, b, w, att_m):
    grid = (NP // BLK,)
    return pl.pallas_call(
        _mid_body,
        grid=grid,
        in_specs=[
            pl.BlockSpec((2, BLK, D), lambda i: (0, i, 0)),
            pl.BlockSpec((1, D), lambda i: (0, 0)),
            pl.BlockSpec((D, D), lambda i: (0, 0)),
            pl.BlockSpec((D, D), lambda i: (0, 0)),
        ],
        out_specs=[
            pl.BlockSpec((BLK, D), lambda i: (i, 0)),
            pl.BlockSpec((BLK, D), lambda i: (i, 0)),
        ],
        out_shape=[
            jax.ShapeDtypeStruct((NP, D), jnp.float32),
            jax.ShapeDtypeStruct((NP, D), jnp.float32),
        ],
    )(acc, b, w, att_m)


def _post_body(p_ref, b_ref, o_ref):
    s = p_ref[0] + p_ref[1] + b_ref[...]
    m = jnp.max(s, axis=1, keepdims=True)
    z = s - m
    lse = jnp.log(jnp.sum(jnp.exp(z), axis=1, keepdims=True))
    o_ref[...] = z - lse


def _tc_post(acc, b):
    grid = (NP // BLK,)
    return pl.pallas_call(
        _post_body,
        grid=grid,
        in_specs=[
            pl.BlockSpec((2, BLK, D), lambda i: (0, i, 0)),
            pl.BlockSpec((1, D), lambda i: (0, 0)),
        ],
        out_specs=pl.BlockSpec((BLK, D), lambda i: (i, 0)),
        out_shape=jax.ShapeDtypeStruct((NP, D), jnp.float32),
    )(acc, b)


def _edge_layer(h_pad, a_srcv, a_dstv, src, dst):
    e = a_srcv[src] + a_dstv[dst]
    e = jnp.where(e >= 0, e, 0.2 * e)
    ee = jnp.exp(e)
    denom = jax.ops.segment_sum(ee, dst, num_segments=NP)
    alpha = ee / (denom[dst] + 1e-16)
    msg = h_pad[src] * alpha[:, None]
    acc = jax.ops.segment_sum(msg, dst, num_segments=NP)
    return acc


def kernel(x, edge_index, W1, att_src1, att_dst1, b1, W2, att_src2, att_dst2, b2):
    loop = jnp.arange(N, dtype=edge_index.dtype)
    src = jnp.concatenate([edge_index[0], loop])
    dst = jnp.concatenate([edge_index[1], loop])
    src = jnp.pad(src, (0, EP - E2), constant_values=N)
    dst = jnp.pad(dst, (0, EP - E2), constant_values=N)

    x_pad = jnp.pad(x, ((0, NP - N), (0, 0)))

    def att_mat(att_s, att_d):
        m = jnp.zeros((D, D), jnp.float32)
        m = m.at[:, 0].set(att_s[0])
        m = m.at[:, 1].set(att_d[0])
        return m

    h1, A1 = _tc_pre(x_pad, W1, att_mat(att_src1, att_dst1))
    acc1 = _edge_layer(h1, A1[:, 0], A1[:, 1], src, dst)
    acc1 = jnp.stack([acc1, jnp.zeros_like(acc1)])

    h2, A2 = _tc_mid(acc1, b1.reshape(1, D), W2, att_mat(att_src2, att_dst2))
    acc2 = _edge_layer(h2, A2[:, 0], A2[:, 1], src, dst)
    acc2 = jnp.stack([acc2, jnp.zeros_like(acc2)])

    out = _tc_post(acc2, b2.reshape(1, D))
    return out[:N]


# speedup vs baseline: 19.1180x; 19.1180x over previous
import functools

import jax
import jax.numpy as jnp
from jax import lax
from jax.experimental import pallas as pl
from jax.experimental.pallas import tpu as pltpu
from jax.experimental.pallas import tpu_sc as plsc

N = 10000
E = 320000
D = 128
NP = 10240
BLK = 1024
E2 = E + N
NC = 2
NS = 16
L = 16
NW = NC * NS
CB = 128
EPW = ((E2 + NW * CB - 1) // (NW * CB)) * CB
EP = EPW * NW
NCH = EPW // CB
TPN = NP // NS


def _pre_body(x_ref, w_ref, att_ref, h_ref, a_ref):
    h = jnp.dot(x_ref[...], w_ref[...], preferred_element_type=jnp.float32)
    h_ref[...] = h
    a_ref[...] = jnp.dot(h, att_ref[...], preferred_element_type=jnp.float32)


def _tc_pre(x_pad, w, att_m):
    grid = (NP // BLK,)
    return pl.pallas_call(
        _pre_body,
        grid=grid,
        in_specs=[
            pl.BlockSpec((BLK, D), lambda i: (i, 0)),
            pl.BlockSpec((D, D), lambda i: (0, 0)),
            pl.BlockSpec((D, D), lambda i: (0, 0)),
        ],
        out_specs=[
            pl.BlockSpec((BLK, D), lambda i: (i, 0)),
            pl.BlockSpec((BLK, D), lambda i: (i, 0)),
        ],
        out_shape=[
            jax.ShapeDtypeStruct((NP, D), jnp.float32),
            jax.ShapeDtypeStruct((NP, D), jnp.float32),
        ],
    )(x_pad, w, att_m)


def _mid_body(p_ref, b_ref, w_ref, att_ref, h_ref, a_ref):
    s = p_ref[0] + p_ref[1] + b_ref[...]
    hin = jnp.maximum(s, 0.0)
    h = jnp.dot(hin, w_ref[...], preferred_element_type=jnp.float32)
    h_ref[...] = h
    a_ref[...] = jnp.dot(h, att_ref[...], preferred_element_type=jnp.float32)


def _tc_mid(acc, b, w, att_m):
    grid = (NP // BLK,)
    return pl.pallas_call(
        _mid_body,
        grid=grid,
        in_specs=[
            pl.BlockSpec((2, BLK, D), lambda i: (0, i, 0)),
            pl.BlockSpec((1, D), lambda i: (0, 0)),
            pl.BlockSpec((D, D), lambda i: (0, 0)),
            pl.BlockSpec((D, D), lambda i: (0, 0)),
        ],
        out_specs=[
            pl.BlockSpec((BLK, D), lambda i: (i, 0)),
            pl.BlockSpec((BLK, D), lambda i: (i, 0)),
        ],
        out_shape=[
            jax.ShapeDtypeStruct((NP, D), jnp.float32),
            jax.ShapeDtypeStruct((NP, D), jnp.float32),
        ],
    )(acc, b, w, att_m)


def _post_body(p_ref, b_ref, o_ref):
    s = p_ref[0] + p_ref[1] + b_ref[...]
    m = jnp.max(s, axis=1, keepdims=True)
    z = s - m
    lse = jnp.log(jnp.sum(jnp.exp(z), axis=1, keepdims=True))
    o_ref[...] = z - lse


def _tc_post(acc, b):
    grid = (NP // BLK,)
    return pl.pallas_call(
        _post_body,
        grid=grid,
        in_specs=[
            pl.BlockSpec((2, BLK, D), lambda i: (0, i, 0)),
            pl.BlockSpec((1, D), lambda i: (0, 0)),
        ],
        out_specs=pl.BlockSpec((BLK, D), lambda i: (i, 0)),
        out_shape=jax.ShapeDtypeStruct((NP, D), jnp.float32),
    )(acc, b)


def _sc_mesh():
    return plsc.VectorSubcoreMesh(
        core_axis_name="c", subcore_axis_name="s",
        num_cores=NC, num_subcores=NS)


def _sc_pass1(src, dst, asrc, adst):
    @functools.partial(
        pl.kernel,
        out_type=[jax.ShapeDtypeStruct((EP,), jnp.float32),
                  jax.ShapeDtypeStruct((NC, NP), jnp.float32)],
        mesh=_sc_mesh(),
        compiler_params=pltpu.CompilerParams(needs_layout_passes=False),
        scratch_types=[
            pltpu.VMEM((NP,), jnp.float32),
            pltpu.VMEM((NP,), jnp.float32),
            pltpu.VMEM((CB,), jnp.int32),
            pltpu.VMEM((CB,), jnp.int32),
            pltpu.VMEM((CB,), jnp.float32),
            pltpu.VMEM((TPN,), jnp.float32),
            pltpu.VMEM_SHARED((NP,), jnp.float32),
        ],
    )
    def k(src_hbm, dst_hbm, asrc_hbm, adst_hbm, ee_hbm, den_hbm,
          asrc_v, adst_v, src_v, dst_v, ee_v, zb_v, den_sh):
        cid = lax.axis_index("c")
        sid = lax.axis_index("s")
        wid = cid * NS + sid

        def zb_body(i, _):
            zb_v[pl.ds(i * L, L)] = jnp.zeros((L,), jnp.float32)
            return 0
        lax.fori_loop(0, TPN // L, zb_body, 0)
        pltpu.sync_copy(zb_v, den_sh.at[pl.ds(sid * TPN, TPN)])
        pltpu.sync_copy(asrc_hbm, asrc_v)
        pltpu.sync_copy(adst_hbm, adst_v)
        plsc.subcore_barrier()

        def chunk(ch, _):
            base = wid * EPW + ch * CB
            pltpu.sync_copy(src_hbm.at[pl.ds(base, CB)], src_v)
            pltpu.sync_copy(dst_hbm.at[pl.ds(base, CB)], dst_v)
            for j in range(CB // L):
                si = src_v[pl.ds(j * L, L)]
                di = dst_v[pl.ds(j * L, L)]
                e = (plsc.load_gather(asrc_v, [si])
                     + plsc.load_gather(adst_v, [di]))
                e = jnp.where(e >= 0.0, e, e * 0.2)
                ee_v[pl.ds(j * L, L)] = jnp.exp(e)
            pltpu.sync_copy(ee_v, ee_hbm.at[pl.ds(base, CB)])
            pltpu.sync_copy(ee_v, den_sh.at[dst_v], add=True)
            return 0
        lax.fori_loop(0, NCH, chunk, 0)
        plsc.subcore_barrier()
        pltpu.sync_copy(den_sh.at[pl.ds(sid * TPN, TPN)],
                        den_hbm.at[cid, pl.ds(sid * TPN, TPN)])

    return k(src, dst, asrc, adst)


def _sc_pass2(src, dst, ee, den, h):
    @functools.partial(
        pl.kernel,
        out_type=jax.ShapeDtypeStruct((NC, NP, D), jnp.float32),
        mesh=_sc_mesh(),
        compiler_params=pltpu.CompilerParams(needs_layout_passes=False),
        scratch_types=[
            pltpu.VMEM((NP,), jnp.float32),
            pltpu.VMEM((NP,), jnp.float32),
            pltpu.VMEM((CB,), jnp.int32),
            pltpu.VMEM((CB,), jnp.int32),
            pltpu.VMEM((CB,), jnp.float32),
            pltpu.VMEM((CB,), jnp.float32),
            pltpu.VMEM((CB, D), jnp.float32),
            pltpu.VMEM_SHARED((NP, D), jnp.float32),
            pltpu.SemaphoreType.DMA,
        ],
    )
    def k(src_hbm, dst_hbm, ee_hbm, den_hbm, h_hbm, acc_hbm,
          rden_v, den2_v, src_v, dst_v, ee_v, al_v, rows_v, acc_sh, sem):
        cid = lax.axis_index("c")
        sid = lax.axis_index("s")
        wid = cid * NS + sid

        pltpu.sync_copy(den_hbm.at[0], rden_v)
        pltpu.sync_copy(den_hbm.at[1], den2_v)

        def rd_body(i, _):
            s = rden_v[pl.ds(i * L, L)] + den2_v[pl.ds(i * L, L)]
            rden_v[pl.ds(i * L, L)] = 1.0 / (s + 1e-16)
            return 0
        lax.fori_loop(0, NP // L, rd_body, 0)

        def z_body(r, _):
            for c in range(D // L):
                rows_v[r, pl.ds(c * L, L)] = jnp.zeros((L,), jnp.float32)
            return 0
        lax.fori_loop(0, CB, z_body, 0)

        def zi_body(kk, _):
            pltpu.sync_copy(rows_v, acc_sh.at[pl.ds(sid * TPN + kk * CB, CB), :])
            return 0
        lax.fori_loop(0, TPN // CB, zi_body, 0)
        plsc.subcore_barrier()

        def chunk(ch, _):
            base = wid * EPW + ch * CB
            pltpu.sync_copy(src_hbm.at[pl.ds(base, CB)], src_v)
            pltpu.sync_copy(dst_hbm.at[pl.ds(base, CB)], dst_v)
            pltpu.sync_copy(ee_hbm.at[pl.ds(base, CB)], ee_v)
            cp = pltpu.async_copy(h_hbm.at[src_v], rows_v, sem)
            for j in range(CB // L):
                di = dst_v[pl.ds(j * L, L)]
                al_v[pl.ds(j * L, L)] = (
                    ee_v[pl.ds(j * L, L)] * plsc.load_gather(rden_v, [di]))
            cp.wait()

            def scale(r, _):
                a = plsc.load_gather(al_v, [jnp.full((L,), r, jnp.int32)])
                for c in range(D // L):
                    rows_v[r, pl.ds(c * L, L)] = rows_v[r, pl.ds(c * L, L)] * a
                return 0
            lax.fori_loop(0, CB, scale, 0)
            pltpu.sync_copy(rows_v, acc_sh.at[dst_v], add=True)
            return 0
        lax.fori_loop(0, NCH, chunk, 0)
        plsc.subcore_barrier()

        def out_body(kk, _):
            r0 = sid * TPN + kk * CB
            pltpu.sync_copy(acc_sh.at[pl.ds(r0, CB), :], rows_v)
            pltpu.sync_copy(rows_v, acc_hbm.at[cid, pl.ds(r0, CB), :])
            return 0
        lax.fori_loop(0, TPN // CB, out_body, 0)

    return k(src, dst, ee, den, h)


def kernel(x, edge_index, W1, att_src1, att_dst1, b1, W2, att_src2, att_dst2, b2):
    loop = jnp.arange(N, dtype=edge_index.dtype)
    src = jnp.concatenate([edge_index[0], loop])
    dst = jnp.concatenate([edge_index[1], loop])
    src = jnp.pad(src, (0, EP - E2), constant_values=N)
    dst = jnp.pad(dst, (0, EP - E2), constant_values=N)

    x_pad = jnp.pad(x, ((0, NP - N), (0, 0)))

    def att_mat(att_s, att_d):
        m = jnp.zeros((D, D), jnp.float32)
        m = m.at[:, 0].set(att_s[0])
        m = m.at[:, 1].set(att_d[0])
        return m

    h1, A1 = _tc_pre(x_pad, W1, att_mat(att_src1, att_dst1))
    ee1, den1 = _sc_pass1(src, dst, A1[:, 0], A1[:, 1])
    acc1 = _sc_pass2(src, dst, ee1, den1, h1)

    h2, A2 = _tc_mid(acc1, b1.reshape(1, D), W2, att_mat(att_src2, att_dst2))
    ee2, den2 = _sc_pass1(src, dst, A2[:, 0], A2[:, 1])
    acc2 = _sc_pass2(src, dst, ee2, den2, h2)

    out = _tc_post(acc2, b2.reshape(1, D))
    return out[:N]
